# baseline (speedup 1.0000x reference)
import functools

import jax
import jax.numpy as jnp
from jax import lax
from jax.experimental import pallas as pl
from jax.experimental.pallas import tpu as pltpu
from jax.experimental.pallas import tpu_sc as plsc

D = 16
CHUNK = 128
NW = 32
BLK = 4


def _make_sc_scatter(n_acc, cpt):
    nblk = cpt // BLK
    assert (nblk - 2) % 3 == 0
    zr = n_acc // 16
    mesh = plsc.VectorSubcoreMesh(core_axis_name="c", subcore_axis_name="s")

    idx_t = pltpu.VMEM((BLK, 2, CHUNK), jnp.int32)
    rows_t = pltpu.VMEM((BLK, CHUNK, D), jnp.float32)

    @functools.partial(
        pl.kernel, mesh=mesh,
        out_type=jax.ShapeDtypeStruct((2, n_acc, D), jnp.float32),
        compiler_params=pltpu.CompilerParams(use_tc_tiling_on_sc=False),
        scratch_types=[
            pltpu.VMEM_SHARED((n_acc, D), jnp.float32),
            idx_t, idx_t, idx_t,
            rows_t, rows_t, rows_t,
            pltpu.SemaphoreType.DMA, pltpu.SemaphoreType.DMA,
            pltpu.SemaphoreType.DMA, pltpu.SemaphoreType.DMA,
            pltpu.SemaphoreType.DMA, pltpu.SemaphoreType.DMA,
        ],
    )
    def sc_scatter(x_hbm, eidx_hbm, zeros_hbm, out_hbm,
                   acc, idxA, idxB, idxC, rowsA, rowsB, rowsC,
                   gA, gB, gC, sA, sB, sC):
        c = lax.axis_index("c")
        s = lax.axis_index("s")
        wid = s * 2 + c
        pltpu.sync_copy(zeros_hbm.at[pl.ds(s * zr, zr)],
                        acc.at[pl.ds(s * zr, zr)])
        plsc.subcore_barrier()

        base = wid * cpt
        bufs = ((idxA, rowsA, gA, sA),
                (idxB, rowsB, gB, sB),
                (idxC, rowsC, gC, sC))

        def load(buf, blk_row):
            pltpu.sync_copy(eidx_hbm.at[pl.ds(blk_row, BLK)], buf[0])

        def fire_g(buf):
            for j in range(BLK):
                pltpu.async_copy(x_hbm.at[buf[0].at[j, 0]], buf[1].at[j],
                                 buf[2])

        def drain_g(buf):
            for j in range(BLK):
                pltpu.make_async_copy(x_hbm.at[buf[0].at[j, 0]],
                                      buf[1].at[j], buf[2]).wait()

        def fire_s(buf):
            for j in range(BLK):
                pltpu.async_copy(buf[1].at[j], acc.at[buf[0].at[j, 1]],
                                 buf[3], add=True)

        def drain_s(buf):
            for j in range(BLK):
                pltpu.make_async_copy(buf[1].at[j],
                                      acc.at[buf[0].at[j, 1]], buf[3]).wait()

        def phase(cur, nxt, nxt_row, first=False):
            if not first:
                drain_s(nxt)
            load(nxt, nxt_row)
            fire_g(nxt)
            drain_g(cur)
            fire_s(cur)

        load(bufs[0], base)
        fire_g(bufs[0])
        phase(bufs[0], bufs[1], base + BLK, first=True)
        phase(bufs[1], bufs[2], base + 2 * BLK, first=True)

        def body(i, carry):
            row = base + (3 * i + 3) * BLK
            phase(bufs[2], bufs[0], row)
            phase(bufs[0], bufs[1], row + BLK)
            phase(bufs[1], bufs[2], row + 2 * BLK)
            return carry

        lax.fori_loop(0, (nblk - 2) // 3, body, 0)
        drain_s(bufs[0])
        drain_s(bufs[1])
        drain_g(bufs[2])

        plsc.subcore_barrier()
        pltpu.sync_copy(acc.at[pl.ds(s * zr, zr)],
                        out_hbm.at[c, pl.ds(s * zr, zr)])

    return sc_scatter


def _dense(parts, x, wrT, wroT, b, rows_blk):
    n = x.shape[0]

    def body(p_ref, x_ref, wr_ref, wo_ref, b_ref, o_ref):
        p = p_ref[0] + p_ref[1]
        acc = jnp.dot(p, wr_ref[...], preferred_element_type=jnp.float32)
        acc += jnp.dot(x_ref[...], wo_ref[...], preferred_element_type=jnp.float32)
        o_ref[...] = jnp.maximum(acc + b_ref[...], 0.0)

    return pl.pallas_call(
        body,
        grid=(n // rows_blk,),
        in_specs=[
            pl.BlockSpec((2, rows_blk, D), lambda i: (0, i, 0)),
            pl.BlockSpec((rows_blk, D), lambda i: (i, 0)),
            pl.BlockSpec((D, D), lambda i: (0, 0)),
            pl.BlockSpec((D, D), lambda i: (0, 0)),
            pl.BlockSpec((1, D), lambda i: (0, 0)),
        ],
        out_specs=pl.BlockSpec((rows_blk, D), lambda i: (i, 0)),
        out_shape=jax.ShapeDtypeStruct((n, D), jnp.float32),
    )(parts, x, wrT, wroT, b)


def kernel(x, edge_index, W1_rel, W1_root, b1, W2_rel, W2_root, b2):
    n = x.shape[0]
    e = edge_index.shape[1]
    n_acc = -(-(n + 1) // CHUNK) * CHUNK

    nblk = -(-(-(-e // (NW * CHUNK))) // BLK)
    while (nblk - 2) % 3:
        nblk += 1
    cpt = nblk * BLK
    e_pad = NW * cpt * CHUNK

    src = edge_index[0].astype(jnp.int32)
    dst = edge_index[1].astype(jnp.int32)
    pad = e_pad - e
    if pad:
        src = jnp.concatenate([src, jnp.zeros((pad,), jnp.int32)])
        dst = jnp.concatenate([dst, jnp.full((pad,), n, jnp.int32)])
    eidx = jnp.stack([src.reshape(NW * cpt, CHUNK),
                      dst.reshape(NW * cpt, CHUNK)], axis=1)
    eidx = jnp.concatenate([eidx, jnp.zeros((BLK, 2, CHUNK), jnp.int32)])
    zeros = jnp.zeros((n_acc, D), jnp.float32)

    sc = _make_sc_scatter(n_acc, cpt)
    rows_blk = 4000

    p1 = sc(x, eidx, zeros)
    h1 = _dense(p1, x, W1_rel.T, W1_root.T, b1.reshape(1, D), rows_blk)
    p2 = sc(h1, eidx, zeros)
    h2 = _dense(p2, h1, W2_rel.T, W2_root.T, b2.reshape(1, D), rows_blk)
    return h2


# speedup vs baseline: 1.3157x; 1.3157x over previous
import functools

import jax
import jax.numpy as jnp
from jax import lax
from jax.experimental import pallas as pl
from jax.experimental.pallas import tpu as pltpu
from jax.experimental.pallas import tpu_sc as plsc

D = 16
CHUNK = 128
NW = 32
BLK = 4
PACK = 128 // D


def _make_sc_scatter(n_acc, nblk):
    assert (nblk - 2) % 3 == 0
    zr = n_acc // 16
    mesh = plsc.VectorSubcoreMesh(core_axis_name="c", subcore_axis_name="s")

    idx_t = pltpu.VMEM((2 * BLK, CHUNK), jnp.int32)
    rows_t = pltpu.VMEM((BLK, CHUNK, D), jnp.float32)

    @functools.partial(
        pl.kernel, mesh=mesh,
        out_type=jax.ShapeDtypeStruct((2, n_acc, D), jnp.float32),
        compiler_params=pltpu.CompilerParams(use_tc_tiling_on_sc=False),
        scratch_types=[
            pltpu.VMEM_SHARED((n_acc, D), jnp.float32),
            idx_t, idx_t, idx_t,
            rows_t, rows_t, rows_t,
            pltpu.SemaphoreType.DMA, pltpu.SemaphoreType.DMA,
            pltpu.SemaphoreType.DMA, pltpu.SemaphoreType.DMA,
            pltpu.SemaphoreType.DMA, pltpu.SemaphoreType.DMA,
        ],
    )
    def sc_scatter(x_hbm, eidx_hbm, zeros_hbm, out_hbm,
                   acc, idxA, idxB, idxC, rowsA, rowsB, rowsC,
                   gA, gB, gC, sA, sB, sC):
        c = lax.axis_index("c")
        s = lax.axis_index("s")
        wid = s * 2 + c
        pltpu.sync_copy(zeros_hbm.at[pl.ds(s * zr, zr)],
                        acc.at[pl.ds(s * zr, zr)])
        plsc.subcore_barrier()

        base = wid * nblk * 2 * BLK
        bufs = ((idxA, rowsA, gA, sA),
                (idxB, rowsB, gB, sB),
                (idxC, rowsC, gC, sC))

        def load(buf, blk_row):
            pltpu.sync_copy(eidx_hbm.at[pl.ds(blk_row, 2 * BLK)], buf[0])

        def fire_g(buf):
            for j in range(BLK):
                pltpu.async_copy(x_hbm.at[buf[0].at[j]], buf[1].at[j], buf[2])

        def drain_g(buf):
            for j in range(BLK):
                pltpu.make_async_copy(x_hbm.at[buf[0].at[j]],
                                      buf[1].at[j], buf[2]).wait()

        def fire_s(buf):
            for j in range(BLK):
                pltpu.async_copy(buf[1].at[j], acc.at[buf[0].at[BLK + j]],
                                 buf[3], add=True)

        def drain_s(buf):
            for j in range(BLK):
                pltpu.make_async_copy(buf[1].at[j],
                                      acc.at[buf[0].at[BLK + j]],
                                      buf[3]).wait()

        def phase(cur, nxt, nxt_row, first=False):
            if not first:
                drain_s(nxt)
            load(nxt, nxt_row)
            fire_g(nxt)
            drain_g(cur)
            fire_s(cur)

        load(bufs[0], base)
        fire_g(bufs[0])
        phase(bufs[0], bufs[1], base + 2 * BLK, first=True)
        phase(bufs[1], bufs[2], base + 4 * BLK, first=True)

        def body(i, carry):
            row = base + (3 * i + 3) * 2 * BLK
            phase(bufs[2], bufs[0], row)
            phase(bufs[0], bufs[1], row + 2 * BLK)
            phase(bufs[1], bufs[2], row + 4 * BLK)
            return carry

        lax.fori_loop(0, (nblk - 2) // 3, body, 0)
        drain_s(bufs[0])
        drain_s(bufs[1])
        drain_g(bufs[2])

        plsc.subcore_barrier()
        pltpu.sync_copy(acc.at[pl.ds(s * zr, zr)],
                        out_hbm.at[c, pl.ds(s * zr, zr)])

    return sc_scatter


def _dense(parts, x128, wr, wo, b, rows_blk):
    m = x128.shape[0]

    def body(p_ref, x_ref, wr_ref, wo_ref, b_ref, o_ref):
        p = p_ref[0] + p_ref[1]
        acc = jnp.dot(p, wr_ref[...], preferred_element_type=jnp.float32)
        acc += jnp.dot(x_ref[...], wo_ref[...], preferred_element_type=jnp.float32)
        o_ref[...] = jnp.maximum(acc + b_ref[...], 0.0)

    return pl.pallas_call(
        body,
        grid=(m // rows_blk,),
        in_specs=[
            pl.BlockSpec((2, rows_blk, 128), lambda i: (0, i, 0)),
            pl.BlockSpec((rows_blk, 128), lambda i: (i, 0)),
            pl.BlockSpec((128, 128), lambda i: (0, 0)),
            pl.BlockSpec((128, 128), lambda i: (0, 0)),
            pl.BlockSpec((1, 128), lambda i: (0, 0)),
        ],
        out_specs=pl.BlockSpec((rows_blk, 128), lambda i: (i, 0)),
        out_shape=jax.ShapeDtypeStruct((m, 128), jnp.float32),
    )(parts, x128, wr, wo, b)


def kernel(x, edge_index, W1_rel, W1_root, b1, W2_rel, W2_root, b2):
    n = x.shape[0]
    e = edge_index.shape[1]
    n_acc = -(-(n + 1) // CHUNK) * CHUNK
    m_acc = n_acc * D // 128
    m_n = n * D // 128

    nblk = -(-(-(-e // (NW * CHUNK))) // BLK)
    while (nblk - 2) % 3:
        nblk += 1
    e_pad = NW * nblk * BLK * CHUNK

    src = edge_index[0].astype(jnp.int32)
    dst = edge_index[1].astype(jnp.int32)
    pad = e_pad - e
    if pad:
        src = jnp.concatenate([src, jnp.zeros((pad,), jnp.int32)])
        dst = jnp.concatenate([dst, jnp.full((pad,), n, jnp.int32)])
    tb = NW * nblk
    eidx = jnp.concatenate([src.reshape(tb, BLK, CHUNK),
                            dst.reshape(tb, BLK, CHUNK)], axis=1)
    eidx = eidx.reshape(tb * 2 * BLK, CHUNK)
    eidx = jnp.concatenate([eidx, jnp.zeros((2 * BLK, CHUNK), jnp.int32)])
    zeros = jnp.zeros((n_acc, D), jnp.float32)

    sc = _make_sc_scatter(n_acc, nblk)
    rows_blk = 3128

    eye = jnp.eye(PACK, dtype=jnp.float32)
    wb1r, wb1o = jnp.kron(eye, W1_rel.T), jnp.kron(eye, W1_root.T)
    wb2r, wb2o = jnp.kron(eye, W2_rel.T), jnp.kron(eye, W2_root.T)
    b1w, b2w = jnp.tile(b1, PACK).reshape(1, 128), jnp.tile(b2, PACK).reshape(1, 128)

    x128 = jnp.pad(x.reshape(m_n, 128), ((0, m_acc - m_n), (0, 0)))

    p1 = sc(x, eidx, zeros)
    h1 = _dense(p1.reshape(2, m_acc, 128), x128, wb1r, wb1o, b1w, rows_blk)
    p2 = sc(h1.reshape(n_acc, D), eidx, zeros)
    h2 = _dense(p2.reshape(2, m_acc, 128), h1, wb2r, wb2o, b2w, rows_blk)
    return h2.reshape(n_acc, D)[:n]


# speedup vs baseline: 1.3836x; 1.0516x over previous
import functools

import jax
import jax.numpy as jnp
from jax import lax
from jax.experimental import pallas as pl
from jax.experimental.pallas import tpu as pltpu
from jax.experimental.pallas import tpu_sc as plsc

D = 16
CHUNK = 128
NW = 32
BLK = 4
PACK = 128 // D


def _make_sc_scatter(n_acc, nb0, nb1):
    assert (nb0 - 2) % 3 == 0 and (nb1 - 2) % 3 == 0
    zr = n_acc // 16
    mesh = plsc.VectorSubcoreMesh(core_axis_name="c", subcore_axis_name="s")

    idx_t = pltpu.VMEM((BLK, CHUNK), jnp.int32)
    rows_t = pltpu.VMEM((BLK, CHUNK, D), jnp.float32)

    @functools.partial(
        pl.kernel, mesh=mesh,
        out_type=jax.ShapeDtypeStruct((2, n_acc, D), jnp.float32),
        compiler_params=pltpu.CompilerParams(use_tc_tiling_on_sc=False),
        scratch_types=[
            pltpu.VMEM_SHARED((n_acc, D), jnp.float32),
            idx_t, idx_t, idx_t,
            idx_t, idx_t, idx_t,
            rows_t, rows_t, rows_t,
            pltpu.SemaphoreType.DMA, pltpu.SemaphoreType.DMA,
            pltpu.SemaphoreType.DMA, pltpu.SemaphoreType.DMA,
            pltpu.SemaphoreType.DMA, pltpu.SemaphoreType.DMA,
        ],
    )
    def sc_scatter(x_hbm, src_hbm, dst_hbm, zeros_hbm, out_hbm,
                   acc, siA, siB, siC, diA, diB, diC, rowsA, rowsB, rowsC,
                   gA, gB, gC, sA, sB, sC):
        c = lax.axis_index("c")
        s = lax.axis_index("s")
        pltpu.sync_copy(zeros_hbm.at[pl.ds(s * zr, zr)],
                        acc.at[pl.ds(s * zr, zr)])
        plsc.subcore_barrier()

        base = (s * (nb0 + nb1) + c * nb0) * BLK
        nphase = jnp.where(c == 0, (nb0 - 2) // 3, (nb1 - 2) // 3)
        bufs = ((siA, diA, rowsA, gA, sA),
                (siB, diB, rowsB, gB, sB),
                (siC, diC, rowsC, gC, sC))

        def load(buf, blk_row):
            pltpu.sync_copy(src_hbm.at[pl.ds(blk_row, BLK)], buf[0])
            pltpu.sync_copy(dst_hbm.at[pl.ds(blk_row, BLK)], buf[1])

        def fire_g(buf):
            for j in range(BLK):
                pltpu.async_copy(x_hbm.at[buf[0].at[j]], buf[2].at[j], buf[3])

        def drain_g(buf):
            for j in range(BLK):
                pltpu.make_async_copy(x_hbm.at[buf[0].at[j]],
                                      buf[2].at[j], buf[3]).wait()

        def fire_s(buf):
            for j in range(BLK):
                pltpu.async_copy(buf[2].at[j], acc.at[buf[1].at[j]],
                                 buf[4], add=True)

        def drain_s(buf):
            for j in range(BLK):
                pltpu.make_async_copy(buf[2].at[j], acc.at[buf[1].at[j]],
                                      buf[4]).wait()

        def phase(cur, nxt, nxt_row, first=False):
            if not first:
                drain_s(nxt)
            load(nxt, nxt_row)
            fire_g(nxt)
            drain_g(cur)
            fire_s(cur)

        load(bufs[0], base)
        fire_g(bufs[0])
        phase(bufs[0], bufs[1], base + BLK, first=True)
        phase(bufs[1], bufs[2], base + 2 * BLK, first=True)

        def body(i, carry):
            row = base + (3 * i + 3) * BLK
            phase(bufs[2], bufs[0], row)
            phase(bufs[0], bufs[1], row + BLK)
            phase(bufs[1], bufs[2], row + 2 * BLK)
            return carry

        lax.fori_loop(0, nphase, body, 0)
        drain_s(bufs[0])
        drain_s(bufs[1])
        drain_g(bufs[2])

        plsc.subcore_barrier()
        pltpu.sync_copy(acc.at[pl.ds(s * zr, zr)],
                        out_hbm.at[c, pl.ds(s * zr, zr)])

    return sc_scatter


def _dense(parts, x128, wr, wo, b, rows_blk):
    m = x128.shape[0]

    def body(p_ref, x_ref, wr_ref, wo_ref, b_ref, o_ref):
        p = p_ref[0] + p_ref[1]
        acc = jnp.dot(p, wr_ref[...], preferred_element_type=jnp.float32)
        acc += jnp.dot(x_ref[...], wo_ref[...], preferred_element_type=jnp.float32)
        o_ref[...] = jnp.maximum(acc + b_ref[...], 0.0)

    return pl.pallas_call(
        body,
        grid=(m // rows_blk,),
        in_specs=[
            pl.BlockSpec((2, rows_blk, 128), lambda i: (0, i, 0)),
            pl.BlockSpec((rows_blk, 128), lambda i: (i, 0)),
            pl.BlockSpec((128, 128), lambda i: (0, 0)),
            pl.BlockSpec((128, 128), lambda i: (0, 0)),
            pl.BlockSpec((1, 128), lambda i: (0, 0)),
        ],
        out_specs=pl.BlockSpec((rows_blk, 128), lambda i: (i, 0)),
        out_shape=jax.ShapeDtypeStruct((m, 128), jnp.float32),
    )(parts, x128, wr, wo, b)


def kernel(x, edge_index, W1_rel, W1_root, b1, W2_rel, W2_root, b2):
    n = x.shape[0]
    e = edge_index.shape[1]
    n_acc = -(-(n + 1) // CHUNK) * CHUNK
    m_acc = n_acc * D // 128
    m_n = n * D // 128

    nbt = 2 * (-(-(-(-e // (NW * CHUNK))) // BLK))
    while True:
        nb0 = -(-(nbt * 245) // 394)
        while (nb0 - 2) % 3:
            nb0 += 1
        nb1 = nbt - nb0
        if nb1 >= 2 and (nb1 - 2) % 3 == 0:
            break
        nbt += 1
    e_pad = 16 * nbt * BLK * CHUNK
    rows_pad = 16 * nbt * BLK + BLK

    src = edge_index[0].astype(jnp.int32)
    dst = edge_index[1].astype(jnp.int32)
    pad = e_pad - e
    srcm = jnp.concatenate(
        [src, jnp.zeros((pad + BLK * CHUNK,), jnp.int32)]).reshape(
            rows_pad, CHUNK)
    dstm = jnp.concatenate(
        [dst, jnp.full((pad,), n, jnp.int32),
         jnp.zeros((BLK * CHUNK,), jnp.int32)]).reshape(rows_pad, CHUNK)
    zeros = jnp.zeros((n_acc, D), jnp.float32)

    sc = _make_sc_scatter(n_acc, nb0, nb1)
    rows_blk = 3128

    eye = jnp.eye(PACK, dtype=jnp.float32)
    wb1r, wb1o = jnp.kron(eye, W1_rel.T), jnp.kron(eye, W1_root.T)
    wb2r, wb2o = jnp.kron(eye, W2_rel.T), jnp.kron(eye, W2_root.T)
    b1w, b2w = jnp.tile(b1, PACK).reshape(1, 128), jnp.tile(b2, PACK).reshape(1, 128)

    x128 = jnp.pad(x.reshape(m_n, 128), ((0, m_acc - m_n), (0, 0)))

    p1 = sc(x, srcm, dstm, zeros)
    h1 = _dense(p1.reshape(2, m_acc, 128), x128, wb1r, wb1o, b1w, rows_blk)
    p2 = sc(h1.reshape(n_acc, D), srcm, dstm, zeros)
    h2 = _dense(p2.reshape(2, m_acc, 128), h1, wb2r, wb2o, b2w, rows_blk)
    return h2.reshape(n_acc, D)[:n]
